# baseline (speedup 1.0000x reference)
import functools

import jax
import jax.numpy as jnp
from jax import lax
from jax.experimental import pallas as pl
from jax.experimental.pallas import tpu as pltpu
from jax.experimental.pallas import tpu_sc as plsc

_H = 16
_NC, _NS = 2, 16
_NW = _NC * _NS
_CHUNK = 128


@functools.lru_cache(maxsize=None)
def _build(B, V):
    b_per_w = B // _NW
    n_ch = b_per_w // _CHUNK
    g_per_ch = _CHUNK // _H
    out_rows_w = b_per_w * _H // 128
    mesh = plsc.VectorSubcoreMesh(core_axis_name="c", subcore_axis_name="s")

    @functools.partial(
        pl.kernel,
        mesh=mesh,
        out_type=jax.ShapeDtypeStruct((B * _H // 128, 128), jnp.float32),
        scratch_types=[
            pltpu.VMEM((b_per_w,), jnp.int32),
            pltpu.VMEM((b_per_w,), jnp.int32),
            pltpu.VMEM((b_per_w, 128), jnp.float32),
            pltpu.VMEM((out_rows_w, 128), jnp.float32),
        ]
        + [pltpu.SemaphoreType.DMA] * n_ch,
        compiler_params=pltpu.CompilerParams(
            use_tc_tiling_on_sc=True, needs_layout_passes=False
        ),
    )
    def gate_kernel(idx_hbm, tab_hbm, out_hbm, idx_v, q_v, rows_v, out_v, *sems):
        wid = lax.axis_index("s") * _NC + lax.axis_index("c")
        base = wid * b_per_w
        pltpu.sync_copy(idx_hbm.at[pl.ds(base, b_per_w)], idx_v)

        def qbody(g, carry):
            iv = idx_v[pl.ds(g * _H, _H)]
            q_v[pl.ds(g * _H, _H)] = lax.shift_right_logical(iv, 3)
            return carry

        lax.fori_loop(0, b_per_w // _H, qbody, 0)

        copies = [
            pltpu.async_copy(
                tab_hbm.at[q_v.at[pl.ds(j * _CHUNK, _CHUNK)]],
                rows_v.at[pl.ds(j * _CHUNK, _CHUNK)],
                sems[j],
            )
            for j in range(n_ch)
        ]

        lane = lax.iota(jnp.int32, _H)
        for j in range(n_ch):
            copies[j].wait()

            def ebody(g, carry, j=j):
                gg = j * g_per_ch + g
                iv = idx_v[pl.ds(gg * _H, _H)]
                colb = (iv & 7) * _H
                rowv = gg * _H + lane
                es = []
                s = None
                for h in range(_H):
                    eh = jnp.exp(plsc.load_gather(rows_v, [rowv, colb + h]))
                    es.append(eh)
                    s = eh if s is None else s + eh
                inv = 1.0 / s
                posb = rowv * _H
                for h in range(_H):
                    pos = posb + h
                    plsc.store_scatter(
                        out_v,
                        [lax.shift_right_logical(pos, 7), pos & 127],
                        es[h] * inv,
                    )
                return carry

            lax.fori_loop(0, g_per_ch, ebody, 0)

        pltpu.sync_copy(out_v, out_hbm.at[pl.ds(wid * out_rows_w, out_rows_w)])

    return gate_kernel


def kernel(user_idx, logits):
    B = user_idx.shape[0]
    V, H = logits.shape
    tab128 = logits.reshape(V * H // 128, 128)
    out = _build(B, V)(user_idx.astype(jnp.int32), tab128)
    return out.reshape(B, H)


# speedup vs baseline: 6.1349x; 6.1349x over previous
import functools

import jax
import jax.numpy as jnp
from jax import lax
from jax.experimental import pallas as pl
from jax.experimental.pallas import tpu as pltpu
from jax.experimental.pallas import tpu_sc as plsc

_H = 16
_NC, _NS = 2, 16
_NW = _NC * _NS
_W = 16


@functools.lru_cache(maxsize=None)
def _build(B, V):
    b_per_w = B // _NW
    n_waves = b_per_w // _W
    mesh = plsc.VectorSubcoreMesh(core_axis_name="c", subcore_axis_name="s")

    @functools.partial(
        pl.kernel,
        mesh=mesh,
        out_type=jax.ShapeDtypeStruct((_H, B), jnp.float32),
        scratch_types=[
            pltpu.VMEM((b_per_w,), jnp.int32),
            pltpu.VMEM((_W, _H, 128), jnp.float32),
            pltpu.VMEM((_W, _H, 128), jnp.float32),
            pltpu.VMEM((_H, b_per_w), jnp.float32),
        ]
        + [pltpu.SemaphoreType.DMA] * 2,
        compiler_params=pltpu.CompilerParams(
            use_tc_tiling_on_sc=True, needs_layout_passes=False
        ),
    )
    def gate_kernel(idx_hbm, tab_hbm, out_hbm, idx_v, wb0, wb1, out_v, *sems):
        wid = lax.axis_index("s") * _NC + lax.axis_index("c")
        base = wid * b_per_w
        pltpu.sync_copy(idx_hbm.at[pl.ds(base, b_per_w)], idx_v)

        uvec = lax.iota(jnp.int32, _W)

        def fire(i, wb, sem):
            iv = idx_v[pl.ds(i * _W, _W)]
            qv = (iv >> 7) * 128
            for u in range(_W):
                pltpu.async_copy(
                    tab_hbm.at[:, pl.ds(pl.multiple_of(qv[u], 128), 128)],
                    wb.at[u],
                    sem,
                )

        def process(i, wb):
            iv = idx_v[pl.ds(i * _W, _W)]
            lvec = iv & 127
            es = []
            s = None
            for h in range(_H):
                hv = jnp.full((_W,), h, jnp.int32)
                eh = jnp.exp(plsc.load_gather(wb, [uvec, hv, lvec]))
                es.append(eh)
                s = eh if s is None else s + eh
            inv = 1.0 / s
            pos = i * _W + uvec
            for h in range(_H):
                hv = jnp.full((_W,), h, jnp.int32)
                plsc.store_scatter(out_v, [hv, pos], es[h] * inv)

        def drain(wb, sem):
            pltpu.make_async_copy(
                tab_hbm.at[:, pl.ds(0, 128)],
                wb.at[0],
                sem,
            ).wait()

        fire(0, wb0, sems[0])

        def body(t, carry):
            i = t * 2
            fire(i + 1, wb1, sems[1])
            for _ in range(_W):
                drain(wb0, sems[0])
            process(i, wb0)

            @pl.when(t + 1 < n_waves // 2)
            def _():
                fire(i + 2, wb0, sems[0])

            for _ in range(_W):
                drain(wb1, sems[1])
            process(i + 1, wb1)
            return carry

        lax.fori_loop(0, n_waves // 2, body, 0)

        pltpu.sync_copy(out_v, out_hbm.at[:, pl.ds(base, b_per_w)])

    return gate_kernel


def kernel(user_idx, logits):
    B = user_idx.shape[0]
    V, H = logits.shape
    out = _build(B, V)(user_idx.astype(jnp.int32), logits.T)
    return out.T
